# baseline (speedup 1.0000x reference)
import functools

import jax
import jax.numpy as jnp
from jax.experimental import pallas as pl
from jax.experimental.pallas import tpu as pltpu

_EPS = 1e-5
_GK_CAP = 3300


def _rnd(n, m=128):
    return -(-n // m) * m


def _linear_body(x_ref, wt_ref, b_ref, o_ref):
    y = jnp.dot(x_ref[...], wt_ref[...], preferred_element_type=jnp.float32)
    o_ref[...] = (y + b_ref[...]).astype(o_ref.dtype)


def _linear(x, wt, b):
    B, out_dim = x.shape[0], wt.shape[1]
    return pl.pallas_call(
        _linear_body,
        out_shape=jax.ShapeDtypeStruct((B, out_dim), jnp.bfloat16),
    )(x, wt, b)


def _conv_body(*refs, offs, G, Cin, Lout, P, Cpb, count, mode):
    if mode == "bn_relu":
        w_ref, x_ref, g_ref, b_ref, m_ref, o_ref, p_ref, acc_ref = refs
    else:
        w_ref, x_ref, o_ref, p_ref = refs
        acc_ref = None
    ngroups = len(offs) // G

    for gidx in range(ngroups):
        for i in range(G):
            off = offs[gidx * G + i]
            p_ref[i * Cin:(i + 1) * Cin, :] = x_ref[:, off:off + Lout]
        part = jnp.dot(w_ref[gidx], p_ref[...],
                       preferred_element_type=jnp.float32)
        if mode == "tanh":
            o_ref[...] = jnp.tanh(part).astype(o_ref.dtype)
            return
        if gidx == 0:
            acc_ref[...] = part
        else:
            acc_ref[...] += part

    acc = acc_ref[...].reshape(Cpb, P, Lout)
    msk = m_ref[...][None]
    inv_n = 1.0 / count
    am = acc * msk
    mean = jnp.sum(am, axis=(1, 2), keepdims=True) * inv_n
    ex2 = jnp.sum(am * acc, axis=(1, 2), keepdims=True) * inv_n
    var = jnp.maximum(ex2 - mean * mean, 0.0)
    z = (acc - mean) * jax.lax.rsqrt(var + _EPS)
    z = z * g_ref[...][..., None] + b_ref[...][..., None]
    o_ref[...] = jnp.maximum(z, 0.0).reshape(Cpb * P, Lout).astype(o_ref.dtype)


def _convt(x, wm, g, b, *, K, stride, padding, cout, mode):
    Cin, N, H, W = x.shape
    Cp = wm.shape[0]
    wmr = wm.reshape(Cp, K, K, Cin)

    if stride == 1:
        q = K - 1 - padding
        P, Ts = 1, K
        OHp = H + 2 * q - K + 1
        ws = wmr.transpose(1, 2, 0, 3)
        offs = None
    else:
        q = K // 2 - 1
        P, Ts = 4, K // 2
        OHp = H - 1 + K // 2
        sub = jnp.stack([wmr[:, K - 1 - a::-2, K - 1 - b::-2, :]
                         for a in (0, 1) for b in (0, 1)], axis=0)
        ws = sub.transpose(2, 3, 1, 0, 4)
    R = P * Cp
    T = Ts * Ts
    ws = ws.reshape(T, R, Cin)

    Hq, Wq = H + 2 * q, W + 2 * q
    if stride == 1:
        offs = [u * Wq + v for u in range(K) for v in range(K)]
    else:
        offs = [(q - u) * Wq + (q - v) for u in range(Ts) for v in range(Ts)]

    G = max(g_ for g_ in range(1, T + 1)
            if T % g_ == 0 and g_ * Cin <= _GK_CAP)
    ngroups = T // G
    wsg = (ws.reshape(ngroups, G, Cp * P, Cin).transpose(0, 2, 1, 3)
           .reshape(ngroups, R, G * Cin).astype(jnp.bfloat16))

    Lvalid = N * Hq * Wq
    Lout = _rnd(Lvalid)
    Lx = _rnd(Lout + max(offs))
    xp = (jnp.pad(x, ((0, 0), (0, 1), (q, q), (q, q)))
          .reshape(Cin, (N + 1) * Hq * Wq)[:, :Lx])
    xf = xp

    body = functools.partial(_conv_body, offs=offs, G=G, Cin=Cin, Lout=Lout,
                             P=P, Cpb=Cp // 2, count=float(P * N * OHp * OHp),
                             mode=mode)
    GK = G * Cin
    scratch = [pltpu.VMEM((GK, Lout), jnp.bfloat16)]
    if mode == "bn_relu":
        idx = jnp.arange(Lout, dtype=jnp.int32)
        mask = (((idx % Wq) < OHp) & ((idx // Wq) % Hq < OHp)
                & (idx < Lvalid)).astype(jnp.float32)[None]
        scratch.append(pltpu.VMEM((R // 2, Lout), jnp.float32))
        out = pl.pallas_call(
            body,
            out_shape=jax.ShapeDtypeStruct((R, Lout), jnp.bfloat16),
            grid=(2,),
            in_specs=[
                pl.BlockSpec((ngroups, R // 2, GK), lambda i: (0, i, 0)),
                pl.BlockSpec((Cin, Lx), lambda i: (0, 0)),
                pl.BlockSpec((Cp // 2, 1), lambda i: (i, 0)),
                pl.BlockSpec((Cp // 2, 1), lambda i: (i, 0)),
                pl.BlockSpec((1, Lout), lambda i: (0, 0)),
            ],
            out_specs=pl.BlockSpec((R // 2, Lout), lambda i: (i, 0)),
            scratch_shapes=scratch,
            compiler_params=pltpu.CompilerParams(
                dimension_semantics=("parallel",)),
        )(wsg, xf, g, b, mask)
    else:
        out = pl.pallas_call(
            body,
            out_shape=jax.ShapeDtypeStruct((R, Lout), jnp.float32),
            grid=(2,),
            in_specs=[
                pl.BlockSpec((ngroups, R // 2, GK), lambda i: (0, i, 0)),
                pl.BlockSpec((Cin, Lx), lambda i: (0, 0)),
            ],
            out_specs=pl.BlockSpec((R // 2, Lout), lambda i: (i, 0)),
            scratch_shapes=scratch,
            compiler_params=pltpu.CompilerParams(
                dimension_semantics=("parallel",)),
        )(wsg, xf)

    if stride == 1:
        y = out[:, :Lvalid].reshape(Cp, N, Hq, Wq)
        return y[:cout, :, :OHp, :OHp]
    y = (out[:, :Lvalid].reshape(Cp, 2, 2, N, Hq, Wq)
         [:cout, :, :, :, :OHp, :OHp])
    return (y.transpose(0, 3, 4, 1, 5, 2)
            .reshape(cout, N, 2 * OHp, 2 * OHp))


def kernel(x, lin_wt, lin_b, wm1, g1, b1, wm2, g2, b2, wm3, g3, b3,
           wm4, g4, b4):
    B = x.shape[0]
    c0 = lin_wt.shape[1] // 25
    c1, c2, c3 = wm1.shape[0], wm2.shape[0], wm3.shape[0]

    h = _linear(x, lin_wt, lin_b)
    h = h.reshape(B, c0, 5, 5).transpose(1, 0, 2, 3)
    h = _convt(h, wm1, g1, b1, K=10, stride=1, padding=1, cout=c1,
               mode="bn_relu")
    h = _convt(h, wm2, g2, b2, K=10, stride=2, padding=0, cout=c2,
               mode="bn_relu")
    h = _convt(h, wm3, g3, b3, K=4, stride=2, padding=0, cout=c3,
               mode="bn_relu")
    h = _convt(h, wm4, g4, b4, K=4, stride=2, padding=0, cout=1,
               mode="tanh")
    return h.transpose(1, 0, 2, 3)


# speedup vs baseline: 1.4684x; 1.4684x over previous
import functools

import jax
import jax.numpy as jnp
from jax.experimental import pallas as pl
from jax.experimental.pallas import tpu as pltpu

_EPS = 1e-5
_GK_CAP = 3300


def _rnd(n, m=128):
    return -(-n // m) * m


def _linear_body(x_ref, wt_ref, b_ref, o_ref):
    y = jnp.dot(x_ref[...], wt_ref[...], preferred_element_type=jnp.float32)
    o_ref[...] = (y + b_ref[...]).astype(o_ref.dtype)


def _linear(x, wt, b):
    B, out_dim = x.shape[0], wt.shape[1]
    return pl.pallas_call(
        _linear_body,
        out_shape=jax.ShapeDtypeStruct((B, out_dim), jnp.bfloat16),
    )(x, wt, b)


def _conv_body(*refs, offs, G, Cin, Lout, P, Cp, count, mode):
    if mode == "bn_relu":
        w_ref, x_ref, g_ref, b_ref, m_ref, o_ref, p_ref, acc_ref = refs
    else:
        w_ref, x_ref, o_ref, p_ref = refs
        acc_ref = None
    ngroups = len(offs) // G

    for gidx in range(ngroups):
        for i in range(G):
            off = offs[gidx * G + i]
            p_ref[i * Cin:(i + 1) * Cin, :] = x_ref[:, off:off + Lout]
        part = jnp.dot(w_ref[gidx], p_ref[...],
                       preferred_element_type=jnp.float32)
        if mode == "tanh":
            o_ref[...] = jnp.tanh(part).astype(o_ref.dtype)
            return
        if gidx == 0:
            acc_ref[...] = part
        else:
            acc_ref[...] += part

    acc = acc_ref[...].reshape(P, Cp, Lout)
    msk = m_ref[...][None]
    inv_n = 1.0 / count
    am = acc * msk
    mean = jnp.sum(am, axis=(0, 2), keepdims=True) * inv_n
    ex2 = jnp.sum(am * acc, axis=(0, 2), keepdims=True) * inv_n
    var = jnp.maximum(ex2 - mean * mean, 0.0)
    z = (acc - mean) * jax.lax.rsqrt(var + _EPS)
    z = z * g_ref[...][None] + b_ref[...][None]
    o_ref[...] = jnp.maximum(z, 0.0).reshape(P * Cp, Lout).astype(o_ref.dtype)


def _convt(x, wm, g, b, *, K, stride, padding, cout, mode):
    Cin, N, H, W = x.shape
    Cp = wm.shape[0]
    wmr = wm.reshape(Cp, K, K, Cin)

    if stride == 1:
        q = K - 1 - padding
        P, Ts = 1, K
        OHp = H + 2 * q - K + 1
        ws = wmr.transpose(1, 2, 0, 3)
        offs = None
    else:
        q = K // 2 - 1
        P, Ts = 4, K // 2
        OHp = H - 1 + K // 2
        sub = jnp.stack([wmr[:, K - 1 - a::-2, K - 1 - b::-2, :]
                         for a in (0, 1) for b in (0, 1)], axis=0)
        ws = sub.transpose(2, 3, 0, 1, 4)
    R = P * Cp
    T = Ts * Ts
    ws = ws.reshape(T, R, Cin)

    Hq, Wq = H + 2 * q, W + 2 * q
    if stride == 1:
        offs = [u * Wq + v for u in range(K) for v in range(K)]
    else:
        offs = [(q - u) * Wq + (q - v) for u in range(Ts) for v in range(Ts)]

    G = max(g_ for g_ in range(1, T + 1)
            if T % g_ == 0 and g_ * Cin <= _GK_CAP)
    ngroups = T // G
    wsg = (ws.reshape(ngroups, G, Cp * P, Cin).transpose(0, 2, 1, 3)
           .reshape(ngroups, R, G * Cin).astype(jnp.bfloat16))

    Lvalid = N * Hq * Wq
    Lout = _rnd(Lvalid)
    Lx = _rnd(Lout + max(offs))
    xf = (jnp.pad(x, ((0, 0), (0, 1), (q, q), (q, q)))
          .reshape(Cin, (N + 1) * Hq * Wq)[:, :Lx])

    body = functools.partial(_conv_body, offs=offs, G=G, Cin=Cin, Lout=Lout,
                             P=P, Cp=Cp, count=float(P * N * OHp * OHp),
                             mode=mode)
    scratch = [pltpu.VMEM((G * Cin, Lout), jnp.bfloat16)]
    if mode == "bn_relu":
        idx = jnp.arange(Lout, dtype=jnp.int32)
        mask = (((idx % Wq) < OHp) & ((idx // Wq) % Hq < OHp)
                & (idx < Lvalid)).astype(jnp.float32)[None]
        scratch.append(pltpu.VMEM((R, Lout), jnp.float32))
        out = pl.pallas_call(
            body,
            out_shape=jax.ShapeDtypeStruct((R, Lout), jnp.bfloat16),
            scratch_shapes=scratch,
        )(wsg, xf, g, b, mask)
    else:
        out = pl.pallas_call(
            body,
            out_shape=jax.ShapeDtypeStruct((R, Lout), jnp.float32),
            scratch_shapes=scratch,
        )(wsg, xf)

    if stride == 1:
        y = out[:, :Lvalid].reshape(Cp, N, Hq, Wq)
        return y[:cout, :, :OHp, :OHp]
    y = out[:, :Lvalid].reshape(2, 2, Cp, N, Hq, Wq)[:, :, :cout, :, :OHp, :OHp]
    return (y.transpose(2, 3, 4, 0, 5, 1)
            .reshape(cout, N, 2 * OHp, 2 * OHp))


def kernel(x, lin_wt, lin_b, wm1, g1, b1, wm2, g2, b2, wm3, g3, b3,
           wm4, g4, b4):
    B = x.shape[0]
    c0 = lin_wt.shape[1] // 25
    c1, c2, c3 = wm1.shape[0], wm2.shape[0], wm3.shape[0]

    h = _linear(x, lin_wt, lin_b)
    h = h.reshape(B, c0, 5, 5).transpose(1, 0, 2, 3)
    h = _convt(h, wm1, g1, b1, K=10, stride=1, padding=1, cout=c1,
               mode="bn_relu")
    h = _convt(h, wm2, g2, b2, K=10, stride=2, padding=0, cout=c2,
               mode="bn_relu")
    h = _convt(h, wm3, g3, b3, K=4, stride=2, padding=0, cout=c3,
               mode="bn_relu")
    h = _convt(h, wm4, g4, b4, K=4, stride=2, padding=0, cout=1,
               mode="tanh")
    return h.transpose(1, 0, 2, 3)
